# baseline (speedup 1.0000x reference)
import jax
import jax.numpy as jnp
from jax import lax
from jax.experimental import pallas as pl
from jax.experimental.pallas import tpu as pltpu

B, S, D_LLM, D_VIT, K, TOPK = 4, 2048, 2048, 1024, 512, 128
TS = 1024


def _fused_kernel(lq_ref, vq_ref, kl_ref, kv_ref, a_ref, bp_ref, x_ref,
                  out_ref, mf_s, pos_s, asel_s, bsel_s):
    b = pl.program_id(0)
    s = pl.program_id(1)

    @pl.when((b == 0) & (s == 0))
    def _route():
        k_ratio = float(D_VIT) / float(D_LLM)
        score = lax.dot_general(
            lq_ref[...], kl_ref[...], (((1,), (1,)), ((), ())),
            preferred_element_type=jnp.float32)
        score = score + k_ratio * lax.dot_general(
            vq_ref[...], kv_ref[...], (((1,), (1,)), ((), ())),
            preferred_element_type=jnp.float32)

        u = lax.bitcast_convert_type(score, jnp.int32)
        key = u ^ ((u >> 31) & jnp.int32(0x7FFFFFFF))

        def count_ge(m):
            return jnp.sum((key >= m).astype(jnp.int32), axis=1,
                           keepdims=True)

        int_min = jnp.full((B, 1), -2147483648, jnp.int32)
        zero = jnp.zeros((B, 1), jnp.int32)
        t = jnp.where(count_ge(zero) >= TOPK, zero, int_min)
        for bit in range(30, -1, -1):
            cand = t | jnp.int32(1 << bit)
            t = jnp.where(count_ge(cand) >= TOPK, cand, t)

        gt = key > t
        eq = key == t
        need = (TOPK - jnp.sum(gt.astype(jnp.int32), axis=1, keepdims=True)
                ).astype(jnp.float32)
        rows = lax.broadcasted_iota(jnp.int32, (K, K), 0)
        cols = lax.broadcasted_iota(jnp.int32, (K, K), 1)
        tri = (rows <= cols).astype(jnp.float32)
        ecs = jnp.dot(eq.astype(jnp.float32), tri,
                      preferred_element_type=jnp.float32)
        mask = jnp.logical_or(gt, jnp.logical_and(eq, ecs <= need))

        mf = mask.astype(jnp.float32)
        mf_s[...] = mf
        pos_s[...] = jnp.dot(mf, tri, preferred_element_type=jnp.float32)

    @pl.when(s == 0)
    def _select():
        rowsel = lax.broadcasted_iota(jnp.int32, (B, 1), 0) == b
        slot_row = (jnp.sum(jnp.where(rowsel, pos_s[...], 0.0), axis=0,
                            keepdims=True) - 1.0).astype(jnp.int32)
        mask_row = jnp.sum(jnp.where(rowsel, mf_s[...], 0.0), axis=0,
                           keepdims=True)
        jrow = lax.broadcasted_iota(jnp.int32, (TOPK, K), 0)
        ptf = jnp.where((slot_row == jrow) & (mask_row > 0.5), 1.0, 0.0)

        asel_s[...] = lax.dot_general(
            a_ref[...], ptf, (((1,), (1,)), ((), ())),
            preferred_element_type=jnp.float32).astype(jnp.bfloat16)
        bsel_s[...] = jnp.dot(ptf, bp_ref[...],
                              preferred_element_type=jnp.float32
                              ).astype(jnp.bfloat16)

    xb = x_ref[0]
    tt = jnp.dot(xb.astype(jnp.bfloat16), asel_s[...],
                 preferred_element_type=jnp.float32)
    out_ref[0] = xb + jnp.dot(tt.astype(jnp.bfloat16), bsel_s[...],
                              preferred_element_type=jnp.float32)


@jax.jit
def kernel(x, llm_query, vit_query, static_keys_llm, static_keys_vit,
           rank_A_pool, rank_B_pool):
    return pl.pallas_call(
        _fused_kernel,
        grid=(B, S // TS),
        in_specs=[
            pl.BlockSpec((B, D_LLM), lambda b, s: (0, 0)),
            pl.BlockSpec((B, D_VIT), lambda b, s: (0, 0)),
            pl.BlockSpec((K, D_LLM), lambda b, s: (0, 0)),
            pl.BlockSpec((K, D_VIT), lambda b, s: (0, 0)),
            pl.BlockSpec((D_LLM, K), lambda b, s: (0, 0)),
            pl.BlockSpec((K, D_LLM), lambda b, s: (0, 0)),
            pl.BlockSpec((1, TS, D_LLM), lambda b, s: (b, s, 0)),
        ],
        out_specs=pl.BlockSpec((1, TS, D_LLM), lambda b, s: (b, s, 0)),
        out_shape=jax.ShapeDtypeStruct((B, S, D_LLM), jnp.float32),
        scratch_shapes=[
            pltpu.VMEM((B, K), jnp.float32),
            pltpu.VMEM((B, K), jnp.float32),
            pltpu.VMEM((D_LLM, TOPK), jnp.bfloat16),
            pltpu.VMEM((TOPK, D_LLM), jnp.bfloat16),
        ],
    )(llm_query, vit_query, static_keys_llm, static_keys_vit,
      rank_A_pool, rank_B_pool, x)


# speedup vs baseline: 1.1210x; 1.1210x over previous
import jax
import jax.numpy as jnp
from jax import lax
from jax.experimental import pallas as pl
from jax.experimental.pallas import tpu as pltpu

B, S, D_LLM, D_VIT, K, TOPK = 4, 2048, 2048, 1024, 512, 128
TS = 1024
TC = 256


def _fused_kernel(lq_ref, vq_ref, kl_ref, kv_ref, a_ref, bp_ref, x_ref,
                  out_ref, mf_s, pos_s, asel_s, bsel_s):
    b = pl.program_id(0)
    s = pl.program_id(1)

    @pl.when((b == 0) & (s == 0))
    def _route():
        k_ratio = float(D_VIT) / float(D_LLM)
        score = lax.dot_general(
            lq_ref[...], kl_ref[...], (((1,), (1,)), ((), ())),
            preferred_element_type=jnp.float32)
        score = score + k_ratio * lax.dot_general(
            vq_ref[...], kv_ref[...], (((1,), (1,)), ((), ())),
            preferred_element_type=jnp.float32)

        u = lax.bitcast_convert_type(score, jnp.int32)
        key = u ^ ((u >> 31) & jnp.int32(0x7FFFFFFF))

        def count_ge(m):
            return jnp.sum((key >= m).astype(jnp.int32), axis=1,
                           keepdims=True)

        int_min = jnp.full((B, 1), -2147483648, jnp.int32)
        zero = jnp.zeros((B, 1), jnp.int32)
        t = jnp.where(count_ge(zero) >= TOPK, zero, int_min)
        for bit in range(30, -1, -1):
            cand = t | jnp.int32(1 << bit)
            t = jnp.where(count_ge(cand) >= TOPK, cand, t)

        gt = key > t
        eq = key == t
        need = (TOPK - jnp.sum(gt.astype(jnp.int32), axis=1, keepdims=True)
                ).astype(jnp.float32)
        rows = lax.broadcasted_iota(jnp.int32, (K, K), 0)
        cols = lax.broadcasted_iota(jnp.int32, (K, K), 1)
        tri = (rows <= cols).astype(jnp.float32)
        ecs = jnp.dot(eq.astype(jnp.float32), tri,
                      preferred_element_type=jnp.float32)
        mask = jnp.logical_or(gt, jnp.logical_and(eq, ecs <= need))

        mf = mask.astype(jnp.float32)
        mf_s[...] = mf
        pos_s[...] = jnp.dot(mf, tri, preferred_element_type=jnp.float32)

    @pl.when(s == 0)
    def _select():
        rowsel = lax.broadcasted_iota(jnp.int32, (B, 1), 0) == b
        slot_row = (jnp.sum(jnp.where(rowsel, pos_s[...], 0.0), axis=0,
                            keepdims=True) - 1.0).astype(jnp.int32)
        mask_row = jnp.sum(jnp.where(rowsel, mf_s[...], 0.0), axis=0,
                           keepdims=True)
        jrow = lax.broadcasted_iota(jnp.int32, (TOPK, K), 0)
        ptf = jnp.where((slot_row == jrow) & (mask_row > 0.5), 1.0, 0.0)

        asel_s[...] = lax.dot_general(
            a_ref[...], ptf, (((1,), (1,)), ((), ())),
            preferred_element_type=jnp.float32).astype(jnp.bfloat16)
        bsel_s[...] = jnp.dot(ptf, bp_ref[...],
                              preferred_element_type=jnp.float32
                              ).astype(jnp.bfloat16)

    for c in range(TS // TC):
        xb = x_ref[0, pl.ds(c * TC, TC), :]
        tt = jnp.dot(xb.astype(jnp.bfloat16), asel_s[...],
                     preferred_element_type=jnp.float32)
        out_ref[0, pl.ds(c * TC, TC), :] = xb + jnp.dot(
            tt.astype(jnp.bfloat16), bsel_s[...],
            preferred_element_type=jnp.float32)


@jax.jit
def kernel(x, llm_query, vit_query, static_keys_llm, static_keys_vit,
           rank_A_pool, rank_B_pool):
    return pl.pallas_call(
        _fused_kernel,
        grid=(B, S // TS),
        in_specs=[
            pl.BlockSpec((B, D_LLM), lambda b, s: (0, 0)),
            pl.BlockSpec((B, D_VIT), lambda b, s: (0, 0)),
            pl.BlockSpec((K, D_LLM), lambda b, s: (0, 0)),
            pl.BlockSpec((K, D_VIT), lambda b, s: (0, 0)),
            pl.BlockSpec((D_LLM, K), lambda b, s: (0, 0)),
            pl.BlockSpec((K, D_LLM), lambda b, s: (0, 0)),
            pl.BlockSpec((1, TS, D_LLM), lambda b, s: (b, s, 0)),
        ],
        out_specs=pl.BlockSpec((1, TS, D_LLM), lambda b, s: (b, s, 0)),
        out_shape=jax.ShapeDtypeStruct((B, S, D_LLM), jnp.float32),
        scratch_shapes=[
            pltpu.VMEM((B, K), jnp.float32),
            pltpu.VMEM((B, K), jnp.float32),
            pltpu.VMEM((D_LLM, TOPK), jnp.bfloat16),
            pltpu.VMEM((TOPK, D_LLM), jnp.bfloat16),
        ],
    )(llm_query, vit_query, static_keys_llm, static_keys_vit,
      rank_A_pool, rank_B_pool, x)
